# baseline (speedup 1.0000x reference)
import jax
import jax.numpy as jnp
from jax import lax
from jax.experimental import pallas as pl
from jax.experimental.pallas import tpu as pltpu
from jax.experimental.pallas import tpu_sc as plsc

_B, _N, _T = 64, 325, 12
_SPACE_D, _DAY_D, _WEEK_D = 64, 32, 32
_DAY_V, _WEEK_V = 288, 7
_OUT_D = 1 + _SPACE_D + _DAY_D + _WEEK_D
_R = _B * _N * _T
_NPAD = 340
_SLOT = 16
_RPAD = _B * _NPAD * _SLOT
_NW = 32
_CP = 20
_CR = _CP * _T
_NG = _CR // 16
_CPB = _NPAD // _CP
_NCHUNK = 2 * _CPB
_LAST_N0 = _N - _CP
_CS = _CP * _SLOT
_XLEN = _CR * 3 + 8


def _sc_body(x_hbm, space_hbm, day_hbm, week_hbm, p_hbm,
             x_v, space_v, day_v, week_v, pbuf_v):
    wid = lax.axis_index("s") * 2 + lax.axis_index("c")

    pltpu.sync_copy(space_hbm, space_v)
    pltpu.sync_copy(day_hbm, day_v)
    pltpu.sync_copy(week_hbm, week_v)

    lanes = lax.iota(jnp.int32, 16)
    lanes3 = lanes * 3
    zeros = lanes * 0
    c128 = zeros + 128

    def chunk(ci, carry):
        b = wid * 2 + ci // _CPB
        n0 = jnp.minimum((ci % _CPB) * _CP, _LAST_N0)
        xoff = (b * _N + n0) * _T * 3
        xal = (xoff // 8) * 8
        skew = xoff - xal
        pltpu.sync_copy(x_hbm.at[pl.ds(xal, _XLEN)], x_v)

        dv, wv = [], []
        for g in range(_NG):
            base = g * 16
            r = base + lanes
            n_i = r // _T
            t_i = r - n_i * _T
            slot_i = n_i * _SLOT + t_i
            idx0 = skew + base * 3 + lanes3
            fvec = plsc.load_gather(x_v, [idx0])
            dvec = plsc.load_gather(x_v, [idx0 + 1]).astype(jnp.int32)
            wvec = plsc.load_gather(x_v, [idx0 + 2]).astype(jnp.int32)
            plsc.store_scatter(pbuf_v, [slot_i, zeros], fvec)
            w31 = plsc.load_gather(week_v, [wvec * _WEEK_D + 31])
            plsc.store_scatter(pbuf_v, [slot_i, c128], w31)
            dv.append(dvec * _DAY_D)
            wv.append(wvec * _WEEK_D)

        for p in range(_CP):
            sb = (n0 + p) * _SPACE_D
            for t in range(_T):
                rr = p * _T + t
                slot = p * _SLOT + t
                g, l = rr // 16, rr % 16
                db = dv[g][l]
                wb = wv[g][l]
                for k in range(4):
                    pbuf_v[slot, pl.ds(1 + 16 * k, 16)] = space_v[pl.ds(sb + 16 * k, 16)]
                for k in range(2):
                    pbuf_v[slot, pl.ds(65 + 16 * k, 16)] = day_v[pl.ds(db + 16 * k, 16)]
                pbuf_v[slot, pl.ds(97, 16)] = week_v[pl.ds(wb, 16)]
                pbuf_v[slot, pl.ds(112, 16)] = week_v[pl.ds(wb + 15, 16)]

        prow0 = (b * _NPAD + n0) * _SLOT
        pltpu.sync_copy(pbuf_v, p_hbm.at[pl.ds(prow0, _CS), :])
        return carry

    lax.fori_loop(0, _NCHUNK, chunk, 0)


def _sc_stage(x_flat, space_flat, day_flat, week_flat):
    mesh = plsc.VectorSubcoreMesh(core_axis_name="c", subcore_axis_name="s")
    return pl.kernel(
        _sc_body,
        mesh=mesh,
        compiler_params=pltpu.CompilerParams(
            needs_layout_passes=False, use_tc_tiling_on_sc=False),
        out_type=jax.ShapeDtypeStruct((_RPAD, 256), jnp.float32),
        scratch_types=[
            pltpu.VMEM((_XLEN,), jnp.float32),
            pltpu.VMEM((_N * _SPACE_D,), jnp.float32),
            pltpu.VMEM((_DAY_V * _DAY_D,), jnp.float32),
            pltpu.VMEM((_WEEK_V * _WEEK_D,), jnp.float32),
            pltpu.VMEM((_CS, 256), jnp.float32),
        ],
    )(x_flat, space_flat, day_flat, week_flat)


def _tc_retile_body(p_ref, out_ref):
    blk = p_ref[...]
    for p in range(_CP):
        out_ref[0, p, :, :] = blk[_SLOT * p:_SLOT * p + _T, 0:_OUT_D]


def _tc_retile(p):
    return pl.pallas_call(
        _tc_retile_body,
        grid=(_B, _CPB),
        in_specs=[pl.BlockSpec((_CS, 256), lambda b, j: (b * _CPB + j, 0))],
        out_specs=pl.BlockSpec((1, _CP, _T, _OUT_D), lambda b, j: (b, j, 0, 0)),
        out_shape=jax.ShapeDtypeStruct((_B, _N, _T, _OUT_D), jnp.float32),
        compiler_params=pltpu.CompilerParams(
            dimension_semantics=("arbitrary", "arbitrary")),
    )(p)


def kernel(X, spaceInfo, dayInfo, weekInfo):
    x_flat = jnp.pad(X.reshape(_R * 3), (0, 8))
    p = _sc_stage(x_flat, spaceInfo.reshape(-1),
                  dayInfo.reshape(-1), weekInfo.reshape(-1))
    return _tc_retile(p)


# speedup vs baseline: 1.2835x; 1.2835x over previous
import jax
import jax.numpy as jnp
from jax import lax
from jax.experimental import pallas as pl
from jax.experimental.pallas import tpu as pltpu
from jax.experimental.pallas import tpu_sc as plsc

_B, _N, _T = 64, 325, 12
_SPACE_D, _DAY_D, _WEEK_D = 64, 32, 32
_DAY_V, _WEEK_V = 288, 7
_OUT_D = 1 + _SPACE_D + _DAY_D + _WEEK_D
_NPAD = 340
_SLOT = 16
_RPAD = _B * _NPAD * _SLOT
_CP = 20
_CR = _CP * _T
_NG = _CR // 16
_CPB = _NPAD // _CP
_NCHUNK = 2 * _CPB
_LAST_N0 = _N - _CP
_CS = _CP * _SLOT


def _sc_body(f_hbm, di_hbm, wi_hbm, space_hbm, day_hbm, week_hbm, p_hbm,
             f_v, di_v, wi_v, space_v, day_v, week_v, pbuf_v):
    wid = lax.axis_index("s") * 2 + lax.axis_index("c")

    pltpu.sync_copy(space_hbm, space_v)
    pltpu.sync_copy(day_hbm, day_v)
    pltpu.sync_copy(week_hbm, week_v)

    lanes = lax.iota(jnp.int32, 16)
    zeros = lanes * 0
    c128 = zeros + 128

    def chunk(ci, carry):
        b = wid * 2 + ci // _CPB
        n0 = jnp.minimum((ci % _CPB) * _CP, _LAST_N0)
        pltpu.sync_copy(f_hbm.at[b, pl.ds(n0, _CP), :], f_v)
        pltpu.sync_copy(di_hbm.at[b, pl.ds(n0, _CP), :], di_v)
        pltpu.sync_copy(wi_hbm.at[b, pl.ds(n0, _CP), :], wi_v)

        dv, wv = [], []
        for g in range(_NG):
            base = g * 16
            r = base + lanes
            n_i = r // _T
            t_i = r - n_i * _T
            slot_i = n_i * _SLOT + t_i
            fvec = plsc.load_gather(f_v, [n_i, t_i])
            dvec = plsc.load_gather(di_v, [n_i, t_i])
            wvec = plsc.load_gather(wi_v, [n_i, t_i])
            plsc.store_scatter(pbuf_v, [slot_i, zeros], fvec)
            w31 = plsc.load_gather(week_v, [wvec * _WEEK_D + 31])
            plsc.store_scatter(pbuf_v, [slot_i, c128], w31)
            dv.append(dvec * _DAY_D)
            wv.append(wvec * _WEEK_D)

        for p in range(_CP):
            sb = (n0 + p) * _SPACE_D
            for t in range(_T):
                rr = p * _T + t
                slot = p * _SLOT + t
                g, l = rr // 16, rr % 16
                db = dv[g][l]
                wb = wv[g][l]
                for k in range(4):
                    pbuf_v[slot, pl.ds(1 + 16 * k, 16)] = space_v[pl.ds(sb + 16 * k, 16)]
                for k in range(2):
                    pbuf_v[slot, pl.ds(65 + 16 * k, 16)] = day_v[pl.ds(db + 16 * k, 16)]
                pbuf_v[slot, pl.ds(97, 16)] = week_v[pl.ds(wb, 16)]
                pbuf_v[slot, pl.ds(112, 16)] = week_v[pl.ds(wb + 15, 16)]

        prow0 = (b * _NPAD + n0) * _SLOT
        pltpu.sync_copy(pbuf_v, p_hbm.at[pl.ds(prow0, _CS), :])
        return carry

    lax.fori_loop(0, _NCHUNK, chunk, 0)


def _sc_stage(featx, dayi, weeki, space_flat, day_flat, week_flat):
    mesh = plsc.VectorSubcoreMesh(core_axis_name="c", subcore_axis_name="s")
    return pl.kernel(
        _sc_body,
        mesh=mesh,
        compiler_params=pltpu.CompilerParams(
            needs_layout_passes=False, use_tc_tiling_on_sc=False),
        out_type=jax.ShapeDtypeStruct((_RPAD, 256), jnp.float32),
        scratch_types=[
            pltpu.VMEM((_CP, _T), jnp.float32),
            pltpu.VMEM((_CP, _T), jnp.int32),
            pltpu.VMEM((_CP, _T), jnp.int32),
            pltpu.VMEM((_N * _SPACE_D,), jnp.float32),
            pltpu.VMEM((_DAY_V * _DAY_D,), jnp.float32),
            pltpu.VMEM((_WEEK_V * _WEEK_D,), jnp.float32),
            pltpu.VMEM((_CS, 256), jnp.float32),
        ],
    )(featx, dayi, weeki, space_flat, day_flat, week_flat)


_PN = 85
_NQ = _NPAD // _PN


def _tc_retile_body(p_ref, out_ref):
    blk = p_ref[...]
    for p in range(_PN):
        out_ref[0, p, :, :] = blk[_SLOT * p:_SLOT * p + _T, 0:_OUT_D]


def _tc_retile(p):
    return pl.pallas_call(
        _tc_retile_body,
        grid=(_B, _NQ),
        in_specs=[pl.BlockSpec((_PN * _SLOT, 256), lambda b, q: (b * _NQ + q, 0))],
        out_specs=pl.BlockSpec((1, _PN, _T, _OUT_D), lambda b, q: (b, q, 0, 0)),
        out_shape=jax.ShapeDtypeStruct((_B, _N, _T, _OUT_D), jnp.float32),
        compiler_params=pltpu.CompilerParams(
            dimension_semantics=("parallel", "arbitrary")),
    )(p)


def kernel(X, spaceInfo, dayInfo, weekInfo):
    featx = X[..., 0]
    dayi = X[..., 1].astype(jnp.int32)
    weeki = X[..., 2].astype(jnp.int32)
    p = _sc_stage(featx, dayi, weeki, spaceInfo.reshape(-1),
                  dayInfo.reshape(-1), weekInfo.reshape(-1))
    return _tc_retile(p)


# speedup vs baseline: 1.3276x; 1.0343x over previous
import jax
import jax.numpy as jnp
from jax import lax
from jax.experimental import pallas as pl
from jax.experimental.pallas import tpu as pltpu
from jax.experimental.pallas import tpu_sc as plsc

_B, _N, _T = 64, 325, 12
_SPACE_D, _DAY_D, _WEEK_D = 64, 32, 32
_DAY_V, _WEEK_V = 288, 7
_OUT_D = 1 + _SPACE_D + _DAY_D + _WEEK_D
_NPAD = 340
_SLOT = 16
_RPAD = _B * _NPAD * _SLOT
_CP = 20
_CR = _CP * _T
_NG = _CR // 16
_CPB = _NPAD // _CP
_NCHUNK = 2 * _CPB
_LAST_N0 = _N - _CP
_CS = _CP * _SLOT


def _sc_body(f_hbm, di_hbm, wi_hbm, space_hbm, day_hbm, week_hbm, p_hbm,
             f_v, di_v, wi_v, space_v, day_v, week_v, pbuf_v):
    wid = lax.axis_index("s") * 2 + lax.axis_index("c")

    pltpu.sync_copy(space_hbm, space_v)
    pltpu.sync_copy(day_hbm, day_v)
    pltpu.sync_copy(week_hbm, week_v)

    lanes = lax.iota(jnp.int32, 16)
    zeros = lanes * 0
    c128 = zeros + 128

    def chunk(ci, carry):
        b = wid * 2 + ci // _CPB
        n0 = jnp.minimum((ci % _CPB) * _CP, _LAST_N0)
        pltpu.sync_copy(f_hbm.at[b, pl.ds(n0, _CP), :], f_v)
        pltpu.sync_copy(di_hbm.at[b, pl.ds(n0, _CP), :], di_v)
        pltpu.sync_copy(wi_hbm.at[b, pl.ds(n0, _CP), :], wi_v)


        dv, wv = [], []
        for g in range(_NG):
            base = g * 16
            r = base + lanes
            n_i = r // _T
            t_i = r - n_i * _T
            slot_i = n_i * _SLOT + t_i
            fvec = plsc.load_gather(f_v, [n_i, t_i])
            dvec = plsc.load_gather(di_v, [n_i, t_i])
            wvec = plsc.load_gather(wi_v, [n_i, t_i])
            plsc.store_scatter(pbuf_v, [slot_i, zeros], fvec)
            w31 = plsc.load_gather(week_v, [wvec * _WEEK_D + 31])
            plsc.store_scatter(pbuf_v, [slot_i, c128], w31)
            dv.append(dvec * _DAY_D)
            wv.append(wvec * _WEEK_D)

        for p in range(_CP):
            sb = (n0 + p) * _SPACE_D
            for t in range(_T):
                rr = p * _T + t
                slot = p * _SLOT + t
                g, l = rr // 16, rr % 16
                db = dv[g][l]
                wb = wv[g][l]
                for k in range(4):
                    pbuf_v[slot, pl.ds(1 + 16 * k, 16)] = space_v[pl.ds(sb + 16 * k, 16)]
                for k in range(2):
                    pbuf_v[slot, pl.ds(65 + 16 * k, 16)] = day_v[pl.ds(db + 16 * k, 16)]
                pbuf_v[slot, pl.ds(97, 16)] = week_v[pl.ds(wb, 16)]
                pbuf_v[slot, pl.ds(112, 16)] = week_v[pl.ds(wb + 15, 16)]

        prow0 = (b * _NPAD + n0) * _SLOT
        pltpu.sync_copy(pbuf_v, p_hbm.at[pl.ds(prow0, _CS), :])
        return carry

    lax.fori_loop(0, _NCHUNK, chunk, 0)


def _sc_stage(featx, dayi, weeki, space_flat, day_flat, week_flat):
    mesh = plsc.VectorSubcoreMesh(core_axis_name="c", subcore_axis_name="s")
    return pl.kernel(
        _sc_body,
        mesh=mesh,
        compiler_params=pltpu.CompilerParams(
            needs_layout_passes=False, use_tc_tiling_on_sc=False,
            skip_device_barrier=True),
        out_type=jax.ShapeDtypeStruct((_RPAD, 256), jnp.float32),
        scratch_types=[
            pltpu.VMEM((_CP, 128), jnp.float32),
            pltpu.VMEM((_CP, 128), jnp.int32),
            pltpu.VMEM((_CP, 128), jnp.int32),
            pltpu.VMEM((_N * _SPACE_D,), jnp.float32),
            pltpu.VMEM((_DAY_V * _DAY_D,), jnp.float32),
            pltpu.VMEM((_WEEK_V * _WEEK_D,), jnp.float32),
            pltpu.VMEM((_CS, 256), jnp.float32),
        ],
    )(featx, dayi, weeki, space_flat, day_flat, week_flat)


_PN = 170
_NQ = _NPAD // _PN


def _tc_retile_body(p_ref, out_ref):
    blk = p_ref[...]
    for p in range(_PN):
        out_ref[0, p, :, :] = blk[_SLOT * p:_SLOT * p + _T, 0:_OUT_D]


def _tc_retile(p):
    return pl.pallas_call(
        _tc_retile_body,
        grid=(_B, _NQ),
        in_specs=[pl.BlockSpec((_PN * _SLOT, 256), lambda b, q: (b * _NQ + q, 0))],
        out_specs=pl.BlockSpec((1, _PN, _T, _OUT_D), lambda b, q: (b, q, 0, 0)),
        out_shape=jax.ShapeDtypeStruct((_B, _N, _T, _OUT_D), jnp.float32),
        compiler_params=pltpu.CompilerParams(
            dimension_semantics=("parallel", "arbitrary")),
    )(p)


def kernel(X, spaceInfo, dayInfo, weekInfo):
    pad = ((0, 0), (0, 3), (0, 128 - _T))
    featx = jnp.pad(X[..., 0], pad)
    dayi = jnp.pad(X[..., 1].astype(jnp.int32), pad)
    weeki = jnp.pad(X[..., 2].astype(jnp.int32), pad)
    p = _sc_stage(featx, dayi, weeki, spaceInfo.reshape(-1),
                  dayInfo.reshape(-1), weekInfo.reshape(-1))
    return _tc_retile(p)
